# baseline (speedup 1.0000x reference)
import functools

import jax
import jax.numpy as jnp
import numpy as np
from jax import lax
from jax.experimental import pallas as pl
from jax.experimental.pallas import tpu as pltpu
from jax.experimental.pallas import tpu_sc as plsc

H = 768
NW = 32
C = 64
RB = 1024


def _gather_body(ids_hbm, word_hbm, out_hbm, ids_v, buf0, buf1,
                 gsem0, gsem1, ssem0, ssem1):
    nc = 2
    wid = lax.axis_index("s") * nc + lax.axis_index("c")
    n_tok = ids_hbm.shape[0]
    tpw = n_tok // NW
    nch = tpw // C
    tok0 = wid * tpw

    pltpu.sync_copy(ids_hbm.at[pl.ds(tok0, tpw)], ids_v)

    bufs = (buf0, buf1)
    gsems = (gsem0, gsem1)
    ssems = (ssem0, ssem1)

    def gather_desc(g, s):
        idx = ids_v.at[pl.ds(g * C, C)]
        return pltpu.make_async_copy(word_hbm.at[idx], bufs[s], gsems[s])

    def scatter_desc(g, s):
        return pltpu.make_async_copy(
            bufs[s], out_hbm.at[pl.ds(tok0 + g * C, C)], ssems[s])

    gather_desc(0, 0).start()
    gather_desc(1, 1).start()

    def round_iter(i, carry):
        g = i * 2
        for s in range(2):
            gather_desc(g + s, s).wait()
            scatter_desc(g + s, s).start()

        @pl.when(g + 2 < nch)
        def _():
            for s in range(2):
                scatter_desc(g + s, s).wait()
                gather_desc(g + 2 + s, s).start()
        return carry

    lax.fori_loop(0, nch // 2, round_iter, 0)
    scatter_desc(nch - 2, 0).wait()
    scatter_desc(nch - 1, 1).wait()


@jax.jit
def _sc_gather(ids, word_emb):
    n = ids.shape[0]
    tpw = n // NW
    mesh = plsc.VectorSubcoreMesh(core_axis_name="c", subcore_axis_name="s")
    run = pl.kernel(
        _gather_body,
        out_type=jax.ShapeDtypeStruct((n, H), jnp.float32),
        mesh=mesh,
        scratch_types=[
            pltpu.VMEM((tpw,), jnp.int32),
            pltpu.VMEM((C, H), jnp.float32),
            pltpu.VMEM((C, H), jnp.float32),
            pltpu.SemaphoreType.DMA,
            pltpu.SemaphoreType.DMA,
            pltpu.SemaphoreType.DMA,
            pltpu.SemaphoreType.DMA,
        ],
    )
    return run(ids, word_emb)


def _ln_body(x_ref, t_ref, b0_ref, bd_ref, gam_ref, bet_ref, o_ref):
    x = x_ref[...]
    t = t_ref[...]
    base = b0_ref[...] + t * bd_ref[...]
    x = x + base
    one = jnp.ones((H, 1), jnp.float32)
    s1 = jax.lax.dot_general(x, one, (((1,), (0,)), ((), ())),
                             precision=lax.Precision.HIGHEST)
    s2 = jax.lax.dot_general(x * x, one, (((1,), (0,)), ((), ())),
                             precision=lax.Precision.HIGHEST)
    mean = s1 * (1.0 / H)
    var = s2 * (1.0 / H) - mean * mean
    inv = lax.rsqrt(var + 1e-5)
    o_ref[...] = ((x - mean) * inv) * gam_ref[...] + bet_ref[...]


def _ln_body_acc(acc_ref, x_ref, t_ref, b0_ref, bd_ref, gam_ref, bet_ref,
                 o_ref):
    del acc_ref
    _ln_body(x_ref, t_ref, b0_ref, bd_ref, gam_ref, bet_ref, o_ref)


def _tc_ln_chunk(acc, rows_k, tf_k, b0, bd, gamma, beta, n_total, blk0):
    m = rows_k.shape[0]
    grid = (m // RB,)
    small = [
        pl.BlockSpec((RB, 1), lambda i: (i, 0)),
        pl.BlockSpec((1, H), lambda i: (0, 0)),
        pl.BlockSpec((1, H), lambda i: (0, 0)),
        pl.BlockSpec((1, H), lambda i: (0, 0)),
        pl.BlockSpec((1, H), lambda i: (0, 0)),
    ]
    rows_spec = pl.BlockSpec((RB, H), lambda i: (i, 0))
    out_spec = pl.BlockSpec((RB, H), lambda i: (i + blk0, 0))
    out_shape = jax.ShapeDtypeStruct((n_total, H), jnp.float32)
    if acc is None:
        return pl.pallas_call(
            _ln_body, grid=grid,
            in_specs=[rows_spec] + small,
            out_specs=out_spec, out_shape=out_shape,
        )(rows_k, tf_k, b0, bd, gamma, beta)
    return pl.pallas_call(
        _ln_body_acc, grid=grid,
        in_specs=[pl.BlockSpec(memory_space=pl.ANY), rows_spec] + small,
        out_specs=out_spec, out_shape=out_shape,
        input_output_aliases={0: 0},
    )(acc, rows_k, tf_k, b0, bd, gamma, beta)


NCHUNK = 2


def kernel(input_ids, token_type_ids, word_emb, type_emb, pos_emb,
           ln_gamma, ln_beta):
    b, s1, s = input_ids.shape
    n = b * s1 * s
    ids = input_ids.reshape(-1).astype(jnp.int32)
    tf = token_type_ids.reshape(-1, 1).astype(jnp.float32)
    b0 = (type_emb[0] + pos_emb[0]).reshape(1, H)
    bd = (type_emb[1] - type_emb[0]).reshape(1, H)
    gam = ln_gamma.astype(jnp.float32).reshape(1, H)
    bet = ln_beta.astype(jnp.float32).reshape(1, H)

    ck = n // NCHUNK
    rows = [_sc_gather(ids[k * ck:(k + 1) * ck], word_emb)
            for k in range(NCHUNK)]
    acc = None
    for k in range(NCHUNK):
        acc = _tc_ln_chunk(acc, rows[k], tf[k * ck:(k + 1) * ck],
                           b0, bd, gam, bet, n, k * (ck // RB))
    return acc.reshape(b, s1, s, H)


# speedup vs baseline: 1.5076x; 1.5076x over previous
import functools

import jax
import jax.numpy as jnp
import numpy as np
from jax import lax
from jax.experimental import pallas as pl
from jax.experimental.pallas import tpu as pltpu
from jax.experimental.pallas import tpu_sc as plsc

H = 768
NW = 32
C = 64
RB = 2048


def _gather_body(ids_hbm, word_hbm, out_hbm, ids_v, buf0, buf1,
                 gsem0, gsem1, ssem0, ssem1):
    nc = 2
    wid = lax.axis_index("s") * nc + lax.axis_index("c")
    n_tok = ids_hbm.shape[0]
    tpw = n_tok // NW
    nch = tpw // C
    tok0 = wid * tpw

    pltpu.sync_copy(ids_hbm.at[pl.ds(tok0, tpw)], ids_v)

    bufs = (buf0, buf1)
    gsems = (gsem0, gsem1)
    ssems = (ssem0, ssem1)

    def gather_desc(g, s):
        idx = ids_v.at[pl.ds(g * C, C)]
        return pltpu.make_async_copy(word_hbm.at[idx], bufs[s], gsems[s])

    def scatter_desc(g, s):
        return pltpu.make_async_copy(
            bufs[s], out_hbm.at[pl.ds(tok0 + g * C, C)], ssems[s])

    gather_desc(0, 0).start()
    gather_desc(1, 1).start()

    def round_iter(i, carry):
        g = i * 2
        for s in range(2):
            gather_desc(g + s, s).wait()
            scatter_desc(g + s, s).start()

        @pl.when(g + 2 < nch)
        def _():
            for s in range(2):
                scatter_desc(g + s, s).wait()
                gather_desc(g + 2 + s, s).start()
        return carry

    lax.fori_loop(0, nch // 2, round_iter, 0)
    scatter_desc(nch - 2, 0).wait()
    scatter_desc(nch - 1, 1).wait()


@jax.jit
def _sc_gather(ids, word_emb):
    n = ids.shape[0]
    tpw = n // NW
    mesh = plsc.VectorSubcoreMesh(core_axis_name="c", subcore_axis_name="s")
    run = pl.kernel(
        _gather_body,
        out_type=jax.ShapeDtypeStruct((n, H), jnp.float32),
        mesh=mesh,
        scratch_types=[
            pltpu.VMEM((tpw,), jnp.int32),
            pltpu.VMEM((C, H), jnp.float32),
            pltpu.VMEM((C, H), jnp.float32),
            pltpu.SemaphoreType.DMA,
            pltpu.SemaphoreType.DMA,
            pltpu.SemaphoreType.DMA,
            pltpu.SemaphoreType.DMA,
        ],
    )
    return run(ids, word_emb)


def _ln_body(x_ref, t_ref, b0_ref, bd_ref, gam_ref, bet_ref, o_ref):
    x = x_ref[...]
    t = t_ref[...]
    base = b0_ref[...] + t * bd_ref[...]
    x = x + base
    s1 = jnp.sum(x, axis=-1, keepdims=True)
    s2 = jnp.sum(x * x, axis=-1, keepdims=True)
    mean = s1 * (1.0 / H)
    var = s2 * (1.0 / H) - mean * mean
    inv = lax.rsqrt(var + 1e-5)
    o_ref[...] = ((x - mean) * inv) * gam_ref[...] + bet_ref[...]


def _ln_body_acc(acc_ref, x_ref, t_ref, b0_ref, bd_ref, gam_ref, bet_ref,
                 o_ref):
    del acc_ref
    _ln_body(x_ref, t_ref, b0_ref, bd_ref, gam_ref, bet_ref, o_ref)


def _tc_ln_chunk(acc, rows_k, tf_k, b0, bd, gamma, beta, n_total, blk0):
    m = rows_k.shape[0]
    grid = (m // RB,)
    small = [
        pl.BlockSpec((RB, 1), lambda i: (i, 0)),
        pl.BlockSpec((1, H), lambda i: (0, 0)),
        pl.BlockSpec((1, H), lambda i: (0, 0)),
        pl.BlockSpec((1, H), lambda i: (0, 0)),
        pl.BlockSpec((1, H), lambda i: (0, 0)),
    ]
    rows_spec = pl.BlockSpec((RB, H), lambda i: (i, 0))
    out_spec = pl.BlockSpec((RB, H), lambda i: (i + blk0, 0))
    out_shape = jax.ShapeDtypeStruct((n_total, H), jnp.float32)
    if acc is None:
        return pl.pallas_call(
            _ln_body, grid=grid,
            in_specs=[rows_spec] + small,
            out_specs=out_spec, out_shape=out_shape,
        )(rows_k, tf_k, b0, bd, gamma, beta)
    return pl.pallas_call(
        _ln_body_acc, grid=grid,
        in_specs=[pl.BlockSpec(memory_space=pl.ANY), rows_spec] + small,
        out_specs=out_spec, out_shape=out_shape,
        input_output_aliases={0: 0},
    )(acc, rows_k, tf_k, b0, bd, gamma, beta)


NCHUNK = 2


def kernel(input_ids, token_type_ids, word_emb, type_emb, pos_emb,
           ln_gamma, ln_beta):
    b, s1, s = input_ids.shape
    n = b * s1 * s
    ids = input_ids.reshape(-1).astype(jnp.int32)
    tf = token_type_ids.reshape(-1, 1).astype(jnp.float32)
    b0 = (type_emb[0] + pos_emb[0]).reshape(1, H)
    bd = (type_emb[1] - type_emb[0]).reshape(1, H)
    gam = ln_gamma.astype(jnp.float32).reshape(1, H)
    bet = ln_beta.astype(jnp.float32).reshape(1, H)

    ck = n // NCHUNK
    rows = [_sc_gather(ids[k * ck:(k + 1) * ck], word_emb)
            for k in range(NCHUNK)]
    acc = None
    for k in range(NCHUNK):
        acc = _tc_ln_chunk(acc, rows[k], tf[k * ck:(k + 1) * ck],
                           b0, bd, gam, bet, n, k * (ck // RB))
    return acc.reshape(b, s1, s, H)


# speedup vs baseline: 1.5611x; 1.0355x over previous
import functools

import jax
import jax.numpy as jnp
import numpy as np
from jax import lax
from jax.experimental import pallas as pl
from jax.experimental.pallas import tpu as pltpu
from jax.experimental.pallas import tpu_sc as plsc

H = 768
NW = 32
C = 32
NSLOT = 4
RB = 2048
NCHUNK = 2


def _gather_body(tok_base, n_chunk, ids_hbm, word_hbm, out_hbm, ids_v,
                 buf0, buf1, buf2, buf3,
                 gsem0, gsem1, gsem2, gsem3, ssem0, ssem1, ssem2, ssem3):
    nc = 2
    wid = lax.axis_index("s") * nc + lax.axis_index("c")
    tpw = n_chunk // NW
    nch = tpw // C
    tok0 = tok_base + wid * tpw

    pltpu.sync_copy(ids_hbm.at[pl.ds(tok0, tpw)], ids_v)

    bufs = (buf0, buf1, buf2, buf3)
    gsems = (gsem0, gsem1, gsem2, gsem3)
    ssems = (ssem0, ssem1, ssem2, ssem3)

    def gather_desc(g, s):
        idx = ids_v.at[pl.ds(g * C, C)]
        return pltpu.make_async_copy(word_hbm.at[idx], bufs[s], gsems[s])

    def scatter_desc(g, s):
        return pltpu.make_async_copy(
            bufs[s], out_hbm.at[pl.ds(tok0 + g * C, C)], ssems[s])

    gather_desc(0, 0).start()
    gather_desc(1, 1).start()

    def round_iter(i, carry):
        for s in range(NSLOT):
            g = i * NSLOT + s
            gather_desc(g, s).wait()
            scatter_desc(g, s).start()

            @pl.when(g + 2 < nch)
            def _():
                @pl.when(g >= 2)
                def _():
                    scatter_desc(g - 2, (s - 2) % NSLOT).wait()
                gather_desc(g + 2, (s + 2) % NSLOT).start()
        return carry

    lax.fori_loop(0, nch // NSLOT, round_iter, 0)
    for g in range(nch - 4, nch):
        scatter_desc(g, g % NSLOT).wait()


@functools.partial(jax.jit, static_argnums=(2, 3))
def _sc_gather(ids, word_emb, tok_base, n_chunk):
    tpw = n_chunk // NW
    mesh = plsc.VectorSubcoreMesh(core_axis_name="c", subcore_axis_name="s")
    run = pl.kernel(
        functools.partial(_gather_body, tok_base, n_chunk),
        out_type=jax.ShapeDtypeStruct((n_chunk, H), jnp.float32),
        mesh=mesh,
        scratch_types=(
            [pltpu.VMEM((tpw,), jnp.int32)]
            + [pltpu.VMEM((C, H), jnp.float32)] * NSLOT
            + [pltpu.SemaphoreType.DMA] * (2 * NSLOT)
        ),
    )
    return run(ids, word_emb)


def _ln_math(x_ref, t_ref, b0_ref, bd_ref, gam_ref, bet_ref, o_ref):
    x = x_ref[...]
    t = t_ref[...].astype(jnp.float32)
    base = b0_ref[...] + t * bd_ref[...]
    x = x + base
    s1 = jnp.sum(x, axis=-1, keepdims=True)
    s2 = jnp.sum(x * x, axis=-1, keepdims=True)
    mean = s1 * (1.0 / H)
    var = s2 * (1.0 / H) - mean * mean
    inv = lax.rsqrt(var + 1e-5)
    o_ref[...] = ((x - mean) * inv) * gam_ref[...] + bet_ref[...]


def _ln_body_acc(acc_ref, x_ref, t_ref, b0_ref, bd_ref, gam_ref, bet_ref,
                 o_ref):
    del acc_ref
    _ln_math(x_ref, t_ref, b0_ref, bd_ref, gam_ref, bet_ref, o_ref)


def _tc_ln_chunk(acc, rows_k, tids, b0, bd, gamma, beta, n_total, blk0):
    m = rows_k.shape[0]
    grid = (m // RB,)
    small = [
        pl.BlockSpec((RB, 1), lambda i: (i + blk0, 0)),
        pl.BlockSpec((1, H), lambda i: (0, 0)),
        pl.BlockSpec((1, H), lambda i: (0, 0)),
        pl.BlockSpec((1, H), lambda i: (0, 0)),
        pl.BlockSpec((1, H), lambda i: (0, 0)),
    ]
    rows_spec = pl.BlockSpec((RB, H), lambda i: (i, 0))
    out_spec = pl.BlockSpec((RB, H), lambda i: (i + blk0, 0))
    out_shape = jax.ShapeDtypeStruct((n_total, H), jnp.float32)
    if acc is None:
        return pl.pallas_call(
            _ln_math, grid=grid,
            in_specs=[rows_spec] + small,
            out_specs=out_spec, out_shape=out_shape,
        )(rows_k, tids, b0, bd, gamma, beta)
    return pl.pallas_call(
        _ln_body_acc, grid=grid,
        in_specs=[pl.BlockSpec(memory_space=pl.ANY), rows_spec] + small,
        out_specs=out_spec, out_shape=out_shape,
        input_output_aliases={0: 0},
    )(acc, rows_k, tids, b0, bd, gamma, beta)


def kernel(input_ids, token_type_ids, word_emb, type_emb, pos_emb,
           ln_gamma, ln_beta):
    b, s1, s = input_ids.shape
    n = b * s1 * s
    ids = input_ids.reshape(-1).astype(jnp.int32)
    tids = token_type_ids.reshape(-1, 1).astype(jnp.int32)
    b0 = (type_emb[0] + pos_emb[0]).reshape(1, H)
    bd = (type_emb[1] - type_emb[0]).reshape(1, H)
    gam = ln_gamma.astype(jnp.float32).reshape(1, H)
    bet = ln_beta.astype(jnp.float32).reshape(1, H)

    ck = n // NCHUNK
    rows = [_sc_gather(ids, word_emb, k * ck, ck) for k in range(NCHUNK)]
    acc = None
    for k in range(NCHUNK):
        acc = _tc_ln_chunk(acc, rows[k], tids, b0, bd, gam, bet,
                           n, k * (ck // RB))
    return acc.reshape(b, s1, s, H)


# speedup vs baseline: 1.5616x; 1.0003x over previous
import functools

import jax
import jax.numpy as jnp
import numpy as np
from jax import lax
from jax.experimental import pallas as pl
from jax.experimental.pallas import tpu as pltpu
from jax.experimental.pallas import tpu_sc as plsc

H = 768
NW = 32
C = 32
NSLOT = 4
RB = 2048
NCHUNK = 2


def _gather_body(tok_base, n_chunk, ids_hbm, word_hbm, out_hbm, ids_v,
                 buf0, buf1, buf2, buf3,
                 gsem0, gsem1, gsem2, gsem3, ssem0, ssem1, ssem2, ssem3):
    nc = 2
    wid = lax.axis_index("s") * nc + lax.axis_index("c")
    tpw = n_chunk // NW
    nch = tpw // C
    tok0 = tok_base + wid * tpw

    pltpu.sync_copy(ids_hbm.at[pl.ds(tok0, tpw)], ids_v)

    bufs = (buf0, buf1, buf2, buf3)
    gsems = (gsem0, gsem1, gsem2, gsem3)
    ssems = (ssem0, ssem1, ssem2, ssem3)

    def gather_desc(g, s):
        idx = ids_v.at[pl.ds(g * C, C)]
        return pltpu.make_async_copy(word_hbm.at[idx], bufs[s], gsems[s])

    def scatter_desc(g, s):
        return pltpu.make_async_copy(
            bufs[s], out_hbm.at[pl.ds(tok0 + g * C, C)], ssems[s])

    gather_desc(0, 0).start()
    gather_desc(1, 1).start()

    def round_iter(i, carry):
        for s in range(NSLOT):
            g = i * NSLOT + s
            gather_desc(g, s).wait()
            scatter_desc(g, s).start()

            @pl.when(g + 2 < nch)
            def _():
                @pl.when(g >= 2)
                def _():
                    scatter_desc(g - 2, (s - 2) % NSLOT).wait()
                gather_desc(g + 2, (s + 2) % NSLOT).start()
        return carry

    lax.fori_loop(0, nch // NSLOT, round_iter, 0)
    for g in range(nch - 4, nch):
        scatter_desc(g, g % NSLOT).wait()


@functools.partial(jax.jit, static_argnums=(2, 3))
def _sc_gather(ids, word_emb, tok_base, n_chunk):
    tpw = n_chunk // NW
    mesh = plsc.VectorSubcoreMesh(core_axis_name="c", subcore_axis_name="s")
    run = pl.kernel(
        functools.partial(_gather_body, tok_base, n_chunk),
        out_type=jax.ShapeDtypeStruct((n_chunk, H), jnp.float32),
        mesh=mesh,
        scratch_types=(
            [pltpu.VMEM((tpw,), jnp.int32)]
            + [pltpu.VMEM((C, H), jnp.float32)] * NSLOT
            + [pltpu.SemaphoreType.DMA] * (2 * NSLOT)
        ),
    )
    return run(ids, word_emb)


def _ln_math(x_ref, t_ref, b0_ref, bd_ref, gam_ref, bet_ref, o_ref):
    x = x_ref[...]
    t = t_ref[...].astype(jnp.float32)
    base = b0_ref[...] + t * bd_ref[...]
    x = x + base
    s1 = jnp.sum(x, axis=-1, keepdims=True)
    s2 = jnp.sum(x * x, axis=-1, keepdims=True)
    mean = s1 * (1.0 / H)
    var = s2 * (1.0 / H) - mean * mean
    inv = lax.rsqrt(var + 1e-5)
    o_ref[...] = ((x - mean) * inv) * gam_ref[...] + bet_ref[...]


def _ln_body_acc(acc_ref, x_ref, t_ref, b0_ref, bd_ref, gam_ref, bet_ref,
                 o_ref):
    del acc_ref
    _ln_math(x_ref, t_ref, b0_ref, bd_ref, gam_ref, bet_ref, o_ref)


def _tc_ln_chunk(acc, rows_k, tids, b0, bd, gamma, beta, n_total, blk0):
    m = rows_k.shape[0]
    grid = (m // RB,)
    small = [
        pl.BlockSpec((RB, 1), lambda i: (i + blk0, 0)),
        pl.BlockSpec((1, H), lambda i: (0, 0)),
        pl.BlockSpec((1, H), lambda i: (0, 0)),
        pl.BlockSpec((1, H), lambda i: (0, 0)),
        pl.BlockSpec((1, H), lambda i: (0, 0)),
    ]
    rows_spec = pl.BlockSpec((RB, H), lambda i: (i, 0))
    out_spec = pl.BlockSpec((RB, H), lambda i: (i + blk0, 0))
    out_shape = jax.ShapeDtypeStruct((n_total, H), jnp.float32)
    if acc is None:
        return pl.pallas_call(
            _ln_math, grid=grid,
            in_specs=[rows_spec] + small,
            out_specs=out_spec, out_shape=out_shape,
        )(rows_k, tids, b0, bd, gamma, beta)
    return pl.pallas_call(
        _ln_body_acc, grid=grid,
        in_specs=[pl.BlockSpec(memory_space=pl.ANY), rows_spec] + small,
        out_specs=out_spec, out_shape=out_shape,
        input_output_aliases={0: 0},
    )(acc, rows_k, tids, b0, bd, gamma, beta)


def kernel(input_ids, token_type_ids, word_emb, type_emb, pos_emb,
           ln_gamma, ln_beta):
    b, s1, s = input_ids.shape
    n = b * s1 * s
    ids = input_ids.reshape(-1).astype(jnp.int32)
    tids = token_type_ids.reshape(-1, 1).astype(jnp.int32)
    b0 = (type_emb[0] + pos_emb[0]).reshape(1, H)
    bd = (type_emb[1] - type_emb[0]).reshape(1, H)
    gam = ln_gamma.astype(jnp.float32).reshape(1, H)
    bet = ln_beta.astype(jnp.float32).reshape(1, H)

    ck = n // NCHUNK
    rows = [_sc_gather(ids[k * ck:(k + 1) * ck], word_emb, 0, ck)
            for k in range(NCHUNK)]
    acc = None
    for k in range(NCHUNK):
        acc = _tc_ln_chunk(acc, rows[k], tids, b0, bd, gam, bet,
                           n, k * (ck // RB))
    return acc.reshape(b, s1, s, H)
